# baseline (speedup 1.0000x reference)
import functools

import jax
import jax.numpy as jnp
from jax import lax
from jax.experimental import pallas as pl
from jax.experimental.pallas import tpu as pltpu
from jax.experimental.pallas import tpu_sc as plsc

N_CAT = 26
VOCAB = 100001
VOCABP = 100008
EMB = 32
N_NUM = 8
B = 1024
L = 50
LP = 56
L8 = 48
ROWS = B * L
SLOTS = N_CAT + N_NUM
NC = 2
NS = 16
NW = NC * NS
BPW = B // NW
GB = 8
NGRP = BPW // GB
HALF = EMB // 2
XW = 64


def _sc_body(table, idx, xs, w, bv, out,
             idxg, xg, fieldbuf, tailbuf, numbuf, wbuf, bbuf,
             ld_sem, gat_sem, st_sem):
    wid = lax.axis_index("s") * NC + lax.axis_index("c")
    b0 = wid * BPW

    pltpu.sync_copy(w, wbuf)
    pltpu.sync_copy(bv, bbuf)

    def slab_copies(jg):
        bg = b0 + jg * GB
        cps = []
        for f in range(N_CAT):
            cps.append(pltpu.make_async_copy(
                idx.at[pl.ds(f * B + bg, GB), :], idxg.at[f], ld_sem))
        for i in range(N_NUM):
            cps.append(pltpu.make_async_copy(
                xs.at[pl.ds(i * B + bg, GB), :],
                xg.at[i, :, pl.ds(0, LP)], ld_sem))
        return cps

    def start_slab(jg):
        for c in slab_copies(jg):
            c.start()

    def wait_slab(jg):
        for c in slab_copies(jg):
            c.wait()

    def cat_store(j, f):
        return pltpu.make_async_copy(
            fieldbuf.at[f, pl.ds(0, L8), :],
            out.at[pl.ds((b0 + j) * L, L8), f, :],
            st_sem)

    def tail_store(j, f):
        return pltpu.make_async_copy(
            tailbuf.at[f],
            out.at[pl.ds((b0 + j) * L + L8, L - L8), f, :],
            st_sem)

    def num_store(j):
        return pltpu.make_async_copy(
            numbuf, out.at[pl.ds((b0 + j) * L, L), pl.ds(N_CAT, N_NUM), :],
            st_sem)

    def wait_stores(j):
        for f in range(N_CAT):
            cat_store(j, f).wait()
            tail_store(j, f).wait()
        num_store(j).wait()

    start_slab(0)

    def chunk(jg, jj):
        j = jg * GB + jj

        @pl.when(j >= 1)
        def _():
            wait_stores(j)

        def gat_copy(f):
            return pltpu.make_async_copy(
                table.at[f].at[idxg.at[f, jj, pl.ds(0, LP)]],
                fieldbuf.at[f],
                gat_sem)

        for f in range(N_CAT):
            gat_copy(f).start()

        def num_rows(r0, nrows):
            for i in range(N_NUM):
                wlo = wbuf[i, pl.ds(0, HALF)]
                whi = wbuf[i, pl.ds(HALF, HALF)]
                blo = bbuf[i, pl.ds(0, HALF)]
                bhi = bbuf[i, pl.ds(HALF, HALF)]
                xv = xg[i, jj, pl.ds(r0, HALF)]
                for r in range(nrows):
                    xs_ = jnp.broadcast_to(xv[r], (HALF,))
                    numbuf[r0 + r, i, pl.ds(0, HALF)] = xs_ * wlo + blo
                    numbuf[r0 + r, i, pl.ds(HALF, HALF)] = xs_ * whi + bhi

        def num_group(h, _):
            num_rows(h * HALF, HALF)
            return 0

        lax.fori_loop(0, 3, num_group, 0)
        num_rows(L8, L - L8)

        for f in range(N_CAT):
            gat_copy(f).wait()

        for f in range(N_CAT):
            for r in range(L - L8):
                tailbuf[f, r, pl.ds(0, HALF)] = \
                    fieldbuf[f, L8 + r, pl.ds(0, HALF)]
                tailbuf[f, r, pl.ds(HALF, HALF)] = \
                    fieldbuf[f, L8 + r, pl.ds(HALF, HALF)]

        for f in range(N_CAT):
            cat_store(j, f).start()
            tail_store(j, f).start()
        num_store(j).start()

    def group(jg, _):
        wait_slab(jg)

        def inner(jj, _):
            chunk(jg, jj)
            return 0

        lax.fori_loop(0, GB, inner, 0)

        @pl.when(jg + 1 < NGRP)
        def _():
            start_slab(jg + 1)

        return 0

    lax.fori_loop(0, NGRP, group, 0)

    wait_stores(BPW - 1)


_sc_call = functools.partial(
    pl.kernel,
    out_type=jax.ShapeDtypeStruct((ROWS, SLOTS, EMB), jnp.float32),
    mesh=plsc.VectorSubcoreMesh(core_axis_name="c", subcore_axis_name="s"),
    compiler_params=pltpu.CompilerParams(use_tc_tiling_on_sc=False),
    scratch_types=[
        pltpu.VMEM((N_CAT, GB, LP), jnp.int32),
        pltpu.VMEM((N_NUM, GB, XW), jnp.float32),
        pltpu.VMEM((N_CAT, LP, EMB), jnp.float32),
        pltpu.VMEM((N_CAT, L - L8, EMB), jnp.float32),
        pltpu.VMEM((L, N_NUM, EMB), jnp.float32),
        pltpu.VMEM((N_NUM, EMB), jnp.float32),
        pltpu.VMEM((N_NUM, EMB), jnp.float32),
        pltpu.SemaphoreType.DMA,
        pltpu.SemaphoreType.DMA,
        pltpu.SemaphoreType.DMA,
    ],
)(_sc_body)


def kernel(cat_features, num_features, mask, cat_tables, num_W, num_b):
    del mask
    tabpad = jnp.pad(cat_tables, ((0, 0), (0, VOCABP - VOCAB), (0, 0)))
    idxpad = jnp.pad(cat_features.astype(jnp.int32),
                     ((0, 0), (0, 0), (0, LP - L))).reshape(N_CAT * B, LP)
    xspad = jnp.pad(num_features,
                    ((0, 0), (0, 0), (0, LP - L))).reshape(N_NUM * B, LP)
    out = _sc_call(tabpad, idxpad, xspad, num_W, num_b)
    return out.reshape(B, L, SLOTS * EMB)


# speedup vs baseline: 1.0541x; 1.0541x over previous
import functools

import jax
import jax.numpy as jnp
from jax import lax
from jax.experimental import pallas as pl
from jax.experimental.pallas import tpu as pltpu
from jax.experimental.pallas import tpu_sc as plsc

N_CAT = 26
VOCAB = 100001
VOCABP = 100008
EMB = 32
N_NUM = 8
B = 1024
L = 50
ROWS = B * L
SLOTS = N_CAT + N_NUM
NC = 2
NS = 16
NW = NC * NS
RPW = ROWS // NW
CH = 32
NCH = RPW // CH
HALF = EMB // 2


def _sc_body(table, idx, xs, w, bv, out,
             idxbuf0, idxbuf1, fieldbuf0, fieldbuf1, numbuf0, numbuf1,
             xbuf, wbuf, bbuf,
             idx_sem0, idx_sem1, gat_sem0, gat_sem1, st_sem0, st_sem1):
    idxbufs = (idxbuf0, idxbuf1)
    fieldbufs = (fieldbuf0, fieldbuf1)
    numbufs = (numbuf0, numbuf1)
    idx_sems = (idx_sem0, idx_sem1)
    gat_sems = (gat_sem0, gat_sem1)
    st_sems = (st_sem0, st_sem1)

    wid = lax.axis_index("s") * NC + lax.axis_index("c")
    row0 = wid * RPW

    for i in range(N_NUM):
        pltpu.sync_copy(xs.at[pl.ds(i * ROWS + row0, RPW)],
                        xbuf.at[pl.ds(i * RPW, RPW)])
    pltpu.sync_copy(w, wbuf)
    pltpu.sync_copy(bv, bbuf)

    def idx_copy(g, b):
        return pltpu.make_async_copy(
            idx.at[:, pl.ds(row0 + g * CH, CH)], idxbufs[b], idx_sems[b])

    def cat_store(g, b, f):
        return pltpu.make_async_copy(
            fieldbufs[b].at[f], out.at[pl.ds(row0 + g * CH, CH), f, :],
            st_sems[b])

    def num_store(g, b):
        return pltpu.make_async_copy(
            numbufs[b], out.at[pl.ds(row0 + g * CH, CH), pl.ds(N_CAT, N_NUM), :],
            st_sems[b])

    def wait_stores(g, b):
        for f in range(N_CAT):
            cat_store(g, b, f).wait()
        num_store(g, b).wait()

    idx_copy(0, 0).start()
    idx_copy(1, 1).start()

    def chunk(g, b):
        @pl.when(g >= 2)
        def _():
            wait_stores(g, b)

        idx_copy(g, b).wait()

        def gat_copy(f):
            return pltpu.make_async_copy(
                table.at[f].at[idxbufs[b].at[f]],
                fieldbufs[b].at[f],
                gat_sems[b])

        for f in range(N_CAT):
            gat_copy(f).start()

        nb = numbufs[b]

        def num_group(h, _):
            r0 = h * HALF
            for i in range(N_NUM):
                wlo = wbuf[i, pl.ds(0, HALF)]
                whi = wbuf[i, pl.ds(HALF, HALF)]
                blo = bbuf[i, pl.ds(0, HALF)]
                bhi = bbuf[i, pl.ds(HALF, HALF)]
                xv = xbuf[pl.ds(i * RPW + g * CH + r0, HALF)]
                for r in range(HALF):
                    xs_ = jnp.broadcast_to(xv[r], (HALF,))
                    nb[r0 + r, i, pl.ds(0, HALF)] = xs_ * wlo + blo
                    nb[r0 + r, i, pl.ds(HALF, HALF)] = xs_ * whi + bhi
            return 0

        lax.fori_loop(0, CH // HALF, num_group, 0)

        for f in range(N_CAT):
            gat_copy(f).wait()

        for f in range(N_CAT):
            cat_store(g, b, f).start()
        num_store(g, b).start()

        @pl.when(g + 2 < NCH)
        def _():
            idx_copy(g + 2, b).start()

    def outer(go, _):
        chunk(2 * go, 0)
        chunk(2 * go + 1, 1)
        return 0

    lax.fori_loop(0, NCH // 2, outer, 0)

    wait_stores(NCH - 2, 0)
    wait_stores(NCH - 1, 1)


_sc_call = functools.partial(
    pl.kernel,
    out_type=jax.ShapeDtypeStruct((ROWS, SLOTS, EMB), jnp.float32),
    mesh=plsc.VectorSubcoreMesh(core_axis_name="c", subcore_axis_name="s"),
    compiler_params=pltpu.CompilerParams(use_tc_tiling_on_sc=False),
    scratch_types=[
        pltpu.VMEM((N_CAT, CH), jnp.int32),
        pltpu.VMEM((N_CAT, CH), jnp.int32),
        pltpu.VMEM((N_CAT, CH, EMB), jnp.float32),
        pltpu.VMEM((N_CAT, CH, EMB), jnp.float32),
        pltpu.VMEM((CH, N_NUM, EMB), jnp.float32),
        pltpu.VMEM((CH, N_NUM, EMB), jnp.float32),
        pltpu.VMEM((N_NUM * RPW,), jnp.float32),
        pltpu.VMEM((N_NUM, EMB), jnp.float32),
        pltpu.VMEM((N_NUM, EMB), jnp.float32),
        pltpu.SemaphoreType.DMA,
        pltpu.SemaphoreType.DMA,
        pltpu.SemaphoreType.DMA,
        pltpu.SemaphoreType.DMA,
        pltpu.SemaphoreType.DMA,
        pltpu.SemaphoreType.DMA,
    ],
)(_sc_body)


def kernel(cat_features, num_features, mask, cat_tables, num_W, num_b):
    del mask
    idx2 = cat_features.astype(jnp.int32).reshape(N_CAT, ROWS)
    xflat = num_features.reshape(N_NUM * ROWS)
    tabpad = jnp.pad(cat_tables, ((0, 0), (0, VOCABP - VOCAB), (0, 0)))
    out = _sc_call(tabpad, idx2, xflat, num_W, num_b)
    return out.reshape(B, L, SLOTS * EMB)
